# baseline (speedup 1.0000x reference)
import functools

import jax
import jax.numpy as jnp
from jax.experimental import pallas as pl
from jax.experimental.pallas import tpu as pltpu

_INVALID0, _INVALID1 = 100, 102
_C = 128
_QB = 2000
_NB = 10
_SR = 16
_NR = _NB * _SR
_B = 8
_T = 8
_K1 = 112
_K = 100
_NEG = -1.0
_BIG = 2**30
_MINF = -1e30


def _q_of(row, lane):
    return (row // _SR) * _QB + (row % _SR) * _C + lane


def _rowmax_kernel(lg_ref, out_ref):
    j = pl.program_id(1)
    lg = lg_ref[0]
    c = jax.lax.broadcasted_iota(jnp.int32, (_QB, _C), 1)
    valid = ((c < _INVALID0) | (c > _INVALID1)) & (c < _C - 1)
    ml = jnp.max(jnp.where(valid, lg, _MINF), axis=1, keepdims=True)
    mlp = jnp.concatenate(
        [ml, jnp.full((_SR * _C - _QB, 1), _MINF, jnp.float32)], axis=0)
    out_ref[0, pl.ds(j * _SR, _SR), :] = mlp.reshape(_SR, _C)


def _select_kernel(ml_ref, obj_ref, kno_ref, ts_ref, lg_hbm, bx_hbm,
                   sc_ref, fl_ref, bo_ref,
                   m_scr, pk_scr, pu_scr, s_scr, cand_bx, bxo_scr,
                   sem1, sem2):
    op = jnp.clip(jnp.exp(-obj_ref[...]), 1e-6, 1.0)
    kp = jnp.clip(jnp.exp(-kno_ref[...]), 1e-6, 1.0)
    pk2 = op * kp
    pu2 = op * jnp.clip(1.0 - kp, 0.0, 1.0) * 15.0
    m2 = jnp.maximum(pk2 * jax.nn.sigmoid(ml_ref[...]), pu2)
    rowp = jax.lax.broadcasted_iota(jnp.int32, (_B, _NR, _C), 1)
    lanep = jax.lax.broadcasted_iota(jnp.int32, (_B, _NR, _C), 2)
    pad = (rowp % _SR == _SR - 1) & (lanep >= _QB - (_SR - 1) * _C)
    m2 = jnp.where(pad, _NEG, m2)
    m_scr[...] = jnp.broadcast_to(m2[:, :, None, :], (_B, _NR, _T, _C))
    pk_scr[...] = jnp.broadcast_to(pk2[:, :, None, :], (_B, _NR, _T, _C))
    pu_scr[...] = jnp.broadcast_to(pu2[:, :, None, :], (_B, _NR, _T, _C))

    rmaxA = jnp.max(m2, axis=2)

    lane128 = jax.lax.broadcasted_iota(jnp.int32, (1, _C), 1)
    laneNR = jax.lax.broadcasted_iota(jnp.int32, (_B, _NR), 1)
    subNR = jax.lax.broadcasted_iota(jnp.int32, (_B, _NR), 0)
    lane8 = jax.lax.broadcasted_iota(jnp.int32, (_B, _C), 1)
    sub8 = jax.lax.broadcasted_iota(jnp.int32, (_B, _C), 0)

    def body(i, carry):
        rmaxA, qacc, pkacc, puacc = carry
        mx = jnp.max(rmaxA, axis=1, keepdims=True)
        srow = jnp.min(jnp.where(rmaxA == mx, laneNR, _BIG),
                       axis=1, keepdims=True)
        for b in range(_B):
            r_b = srow[b, 0]
            mx_b = mx[b, 0]
            row = m_scr[b, pl.ds(r_b, 1)][0][0:1, :]
            l_b = jnp.min(jnp.where(row == mx_b, lane128, _BIG))
            hit1 = lane128 == l_b
            pkrow = pk_scr[b, pl.ds(r_b, 1)][0][0:1, :]
            purow = pu_scr[b, pl.ds(r_b, 1)][0][0:1, :]
            pk_b = jnp.sum(jnp.where(hit1, pkrow, 0.0))
            pu_b = jnp.sum(jnp.where(hit1, purow, 0.0))
            newrow = jnp.where(hit1, _NEG, row)
            m_scr[b, pl.ds(r_b, 1)] = jnp.broadcast_to(
                newrow, (_T, _C))[None]
            rm_b = jnp.max(newrow)
            rmaxA = jnp.where((laneNR == r_b) & (subNR == b), rm_b, rmaxA)
            q_b = _q_of(r_b, l_b)
            sel = (lane8 == i) & (sub8 == b)
            qacc = jnp.where(sel, q_b, qacc)
            pkacc = jnp.where(sel, pk_b, pkacc)
            puacc = jnp.where(sel, pu_b, puacc)
            pltpu.make_async_copy(lg_hbm.at[b, q_b],
                                  s_scr.at[b, i, 0], sem1).start()
            pltpu.make_async_copy(bx_hbm.at[b, q_b],
                                  cand_bx.at[b, i, 0], sem2).start()
        return rmaxA, qacc, pkacc, puacc

    qacc0 = jnp.zeros((_B, _C), jnp.int32)
    facc0 = jnp.zeros((_B, _C), jnp.float32)
    rmaxA, qacc, pkacc, puacc = jax.lax.fori_loop(
        0, _K1, body, (rmaxA, qacc0, facc0, facc0))

    pltpu.make_async_copy(lg_hbm.at[:, 0:_K1, :],
                          s_scr.at[:, :, 0, :], sem1).wait()
    pltpu.make_async_copy(bx_hbm.at[:, 0:_K1, :],
                          cand_bx.at[:, :, 0, :], sem2).wait()

    lgt = s_scr[...]
    c4 = jax.lax.broadcasted_iota(jnp.int32, (_B, _K1, _T, _C), 3)
    valid = ((c4 < _INVALID0) | (c4 > _INVALID1)) & (c4 < _C - 1)
    pks = pkacc[:, :_K1, None, None]
    pus = puacc[:, :_K1, None, None]
    st = jnp.where(valid, pks * jax.nn.sigmoid(lgt), 0.0)
    st = jnp.where(c4 == _C - 1, pus, st)
    s_scr[...] = st

    r3 = jnp.max(st[:, :, 0:1, :], axis=(2, 3))
    rmax3 = jnp.concatenate(
        [r3, jnp.full((_B, _C - _K1), _NEG, jnp.float32)], axis=1)

    sub81 = jax.lax.broadcasted_iota(jnp.int32, (_B, 1), 0)

    def body3(t, carry):
        rmax3, sco, flo = carry
        mx = jnp.max(rmax3, axis=1, keepdims=True)
        rsel = jnp.min(jnp.where(rmax3 == mx, lane8, _BIG),
                       axis=1, keepdims=True)
        govec = jnp.zeros((_B, 1), jnp.int32)
        for b in range(_B):
            r_b = rsel[b, 0]
            mx_b = mx[b, 0]
            row = s_scr[b, pl.ds(r_b, 1)][0][0:1, :]
            c_b = jnp.min(jnp.where(row == mx_b, lane128, _BIG))
            hit1 = lane128 == c_b
            newrow = jnp.where(hit1, _NEG, row)
            s_scr[b, pl.ds(r_b, 1)] = jnp.broadcast_to(
                newrow, (_T, _C))[None]
            rm_b = jnp.max(newrow)
            rmax3 = jnp.where((lane8 == r_b) & (sub8 == b), rm_b, rmax3)
            q_b = jnp.sum(jnp.where((lane8[0:1] == r_b), qacc[b:b + 1], 0))
            govec = jnp.where(sub81 == b, q_b * _C + c_b, govec)
            brow = cand_bx[b, pl.ds(r_b, 1)][0][0:1, :]
            bxo_scr[b, pl.ds(t, 1)] = jnp.broadcast_to(brow, (_T, 4))[None]
        sco = jnp.where(lane8 == t, mx, sco)
        flo = jnp.where(lane8 == t, govec, flo)
        return rmax3, sco, flo

    sco0 = jnp.zeros((_B, _C), jnp.float32)
    flo0 = jnp.zeros((_B, _C), jnp.int32)
    _, sco, flo = jax.lax.fori_loop(0, _K, body3, (rmax3, sco0, flo0))

    ts = ts_ref[...].astype(jnp.float32)
    hpx = ts[:, 0:1, None]
    wpx = ts[:, 1:2, None]
    bxo = bxo_scr[:, :, 0, :]
    cx = bxo[:, :, 0:1]
    cy = bxo[:, :, 1:2]
    w = bxo[:, :, 2:3]
    h = bxo[:, :, 3:4]
    xyxy = jnp.concatenate(
        [(cx - 0.5 * w) * wpx, (cy - 0.5 * h) * hpx,
         (cx + 0.5 * w) * wpx, (cy + 0.5 * h) * hpx], axis=2)

    sc_ref[...] = sco
    fl_ref[...] = flo
    bo_ref[...] = xyxy


def _pad_energy(e):
    e3 = e.reshape(_B, _NB, _QB)
    e3 = jnp.pad(e3, ((0, 0), (0, 0), (0, _SR * _C - _QB)))
    return e3.reshape(_B, _NR, _C)


@jax.jit
def kernel(class_logits, objectness_energy, knownness_energy, pred_boxes,
           target_sizes):
    B, Q, C = class_logits.shape
    assert C == _C and Q == _QB * _NB and B == _B

    ml = pl.pallas_call(
        _rowmax_kernel,
        grid=(B, _NB),
        in_specs=[pl.BlockSpec((1, _QB, _C), lambda b, j: (b, j, 0))],
        out_specs=pl.BlockSpec((1, _NR, _C), lambda b, j: (b, 0, 0)),
        out_shape=jax.ShapeDtypeStruct((B, _NR, _C), jnp.float32),
    )(class_logits)

    obj3 = _pad_energy(objectness_energy)
    kno3 = _pad_energy(knownness_energy)

    sco, flo, bxo = pl.pallas_call(
        _select_kernel,
        in_specs=[
            pl.BlockSpec((B, _NR, _C), lambda: (0, 0, 0)),
            pl.BlockSpec((B, _NR, _C), lambda: (0, 0, 0)),
            pl.BlockSpec((B, _NR, _C), lambda: (0, 0, 0)),
            pl.BlockSpec((B, 2), lambda: (0, 0)),
            pl.BlockSpec(memory_space=pl.ANY),
            pl.BlockSpec(memory_space=pl.ANY),
        ],
        out_specs=[
            pl.BlockSpec((B, _C), lambda: (0, 0)),
            pl.BlockSpec((B, _C), lambda: (0, 0)),
            pl.BlockSpec((B, _C, 4), lambda: (0, 0, 0)),
        ],
        out_shape=[
            jax.ShapeDtypeStruct((B, _C), jnp.float32),
            jax.ShapeDtypeStruct((B, _C), jnp.int32),
            jax.ShapeDtypeStruct((B, _C, 4), jnp.float32),
        ],
        scratch_shapes=[
            pltpu.VMEM((_B, _NR, _T, _C), jnp.float32),
            pltpu.VMEM((_B, _NR, _T, _C), jnp.float32),
            pltpu.VMEM((_B, _NR, _T, _C), jnp.float32),
            pltpu.VMEM((_B, _K1, _T, _C), jnp.float32),
            pltpu.VMEM((_B, _K1, _T, 4), jnp.float32),
            pltpu.VMEM((_B, _C, _T, 4), jnp.float32),
            pltpu.SemaphoreType.DMA,
            pltpu.SemaphoreType.DMA,
        ],
    )(ml, obj3, kno3, target_sizes, class_logits, pred_boxes)

    scores = sco[:, :_K]
    flat = flo[:, :_K]
    labels = flat % _C
    boxes = bxo[:, :_K, :]
    return scores, labels, boxes


# speedup vs baseline: 1.9990x; 1.9990x over previous
import functools

import jax
import jax.numpy as jnp
from jax import lax
from jax.experimental import pallas as pl
from jax.experimental.pallas import tpu as pltpu
from jax.experimental.pallas import tpu_sc as plsc

_INVALID0, _INVALID1 = 100, 102
_C = 128
_QB = 2000
_NB = 10
_B = 8
_Q = _QB * _NB
_K1 = 112
_K = 100
_NEG = -1.0
_BIG = 2**30
_NW = 32
_GN = _B * _C
_GW = _GN // _NW


def _rowmax_kernel(lg_ref, out_ref):
    j = pl.program_id(1)
    lg = lg_ref[0]
    c = jax.lax.broadcasted_iota(jnp.int32, (_QB, _C), 1)
    valid = ((c < _INVALID0) | (c > _INVALID1)) & (c < _C - 1)
    ml = jnp.max(jnp.where(valid, lg, -1e30), axis=1)
    out_ref[0, pl.ds(j, 1), :] = ml[None, :]


def _select_kernel(ml_ref, obj_ref, kno_ref, bx_hbm,
                   q_ref, pk_ref, pu_ref, rid_ref, cbx_ref, sem2):
    ml2 = ml_ref[...]
    op = jnp.clip(jnp.exp(-obj_ref[...]), 1e-6, 1.0)
    kp = jnp.clip(jnp.exp(-kno_ref[...]), 1e-6, 1.0)
    pk2 = op * kp
    pu2 = op * jnp.clip(1.0 - kp, 0.0, 1.0) * 15.0
    m2 = jnp.maximum(pk2 * jax.nn.sigmoid(ml2), pu2)

    flat3 = (jax.lax.broadcasted_iota(jnp.int32, (_B, _NB, _QB), 1) * _QB
             + jax.lax.broadcasted_iota(jnp.int32, (_B, _NB, _QB), 2))
    lane8 = jax.lax.broadcasted_iota(jnp.int32, (_B, _C), 1)

    def body(i, carry):
        m2, qacc, pkacc, puacc = carry
        mx = jnp.max(m2, axis=(1, 2), keepdims=True)
        hit = m2 == mx
        qstar = jnp.min(jnp.where(hit, flat3, _BIG),
                        axis=(1, 2), keepdims=True)
        pick = hit & (flat3 == qstar)
        pkv = jnp.sum(jnp.where(pick, pk2, 0.0), axis=(1, 2), keepdims=True)
        puv = jnp.sum(jnp.where(pick, pu2, 0.0), axis=(1, 2), keepdims=True)
        onec = lane8 == i
        qacc = qacc + jnp.where(onec, qstar[:, :, 0], 0)
        pkacc = pkacc + jnp.where(onec, pkv[:, :, 0], 0.0)
        puacc = puacc + jnp.where(onec, puv[:, :, 0], 0.0)
        for b in range(_B):
            q = qstar[b, 0, 0]
            pltpu.make_async_copy(bx_hbm.at[b, q], cbx_ref.at[b, i],
                                  sem2).start()
        m2 = jnp.where(pick, _NEG, m2)
        return m2, qacc, pkacc, puacc

    qacc0 = jnp.zeros((_B, _C), jnp.int32)
    facc0 = jnp.zeros((_B, _C), jnp.float32)
    _, qacc, pkacc, puacc = jax.lax.fori_loop(
        0, _K1, body, (m2, qacc0, facc0, facc0))

    pltpu.make_async_copy(bx_hbm.at[:, 0:_K1, :], cbx_ref, sem2).wait()

    sub8 = jax.lax.broadcasted_iota(jnp.int32, (_B, _C), 0)
    q_ref[...] = qacc
    pk_ref[...] = pkacc
    pu_ref[...] = puacc
    rid_ref[...] = qacc + sub8 * _Q


def _sc_gather_kernel(rid_hbm, lgflat_hbm, out_hbm, idx_v, rows_v, sem):
    wid = lax.axis_index("s") * 2 + lax.axis_index("c")
    base = wid * _GW
    pltpu.sync_copy(rid_hbm.at[pl.ds(base, _GW)], idx_v)
    pltpu.async_copy(lgflat_hbm.at[idx_v], rows_v, sem).wait()
    pltpu.sync_copy(rows_v, out_hbm.at[pl.ds(base, _GW)])


def _finalize_kernel(lgc_ref, q_ref, pk_ref, pu_ref, cbx_ref, ts_ref,
                     sc_ref, fl_ref, bo_ref):
    lgc = lgc_ref[:, 0:_K1, :]
    qacc = q_ref[...]
    c3 = jax.lax.broadcasted_iota(jnp.int32, (_B, _K1, _C), 2)
    valid = ((c3 < _INVALID0) | (c3 > _INVALID1)) & (c3 < _C - 1)
    pks = pk_ref[...][:, :_K1, None]
    pus = pu_ref[...][:, :_K1, None]
    qss = qacc[:, :_K1, None]
    s = jnp.where(valid, pks * jax.nn.sigmoid(lgc), 0.0)
    s = jnp.where(c3 == _C - 1, pus, s)
    g = qss * _C + c3
    sub = jax.lax.broadcasted_iota(jnp.int32, (_B, _K1, _C), 1)
    sub_b = jax.lax.broadcasted_iota(jnp.int32, (_B, _K1, 1), 1)
    out_b = jax.lax.broadcasted_iota(jnp.int32, (_B, _C, 1), 1)
    lane8 = jax.lax.broadcasted_iota(jnp.int32, (_B, _C), 1)
    boxc = cbx_ref[...]

    def body3(t, carry):
        s, sco, flo, bxo = carry
        mx = jnp.max(s, axis=(1, 2), keepdims=True)
        hit = s == mx
        gm = jnp.min(jnp.where(hit, g, _BIG), axis=(1, 2), keepdims=True)
        pick = hit & (g == gm)
        r = jnp.min(jnp.where(pick, sub, _BIG), axis=(1, 2), keepdims=True)
        onec = lane8 == t
        sco = sco + jnp.where(onec, mx[:, :, 0], 0.0)
        flo = flo + jnp.where(onec, gm[:, :, 0], 0)
        brow = jnp.sum(jnp.where(sub_b == r, boxc, 0.0), axis=1,
                       keepdims=True)
        bxo = bxo + jnp.where(out_b == t, brow, 0.0)
        s = jnp.where(pick, _NEG, s)
        return s, sco, flo, bxo

    sco0 = jnp.zeros((_B, _C), jnp.float32)
    flo0 = jnp.zeros((_B, _C), jnp.int32)
    bxo0 = jnp.zeros((_B, _C, 4), jnp.float32)
    _, sco, flo, bxo = jax.lax.fori_loop(0, _K, body3, (s, sco0, flo0, bxo0))

    ts = ts_ref[...].astype(jnp.float32)
    hpx = ts[:, 0:1, None]
    wpx = ts[:, 1:2, None]
    cx = bxo[:, :, 0:1]
    cy = bxo[:, :, 1:2]
    w = bxo[:, :, 2:3]
    h = bxo[:, :, 3:4]
    xyxy = jnp.concatenate(
        [(cx - 0.5 * w) * wpx, (cy - 0.5 * h) * hpx,
         (cx + 0.5 * w) * wpx, (cy + 0.5 * h) * hpx], axis=2)

    sc_ref[...] = sco
    fl_ref[...] = flo
    bo_ref[...] = xyxy


@jax.jit
def kernel(class_logits, objectness_energy, knownness_energy, pred_boxes,
           target_sizes):
    B, Q, C = class_logits.shape
    assert C == _C and Q == _Q and B == _B

    ml = pl.pallas_call(
        _rowmax_kernel,
        grid=(B, _NB),
        in_specs=[pl.BlockSpec((1, _QB, _C), lambda b, j: (b, j, 0))],
        out_specs=pl.BlockSpec((1, _NB, _QB), lambda b, j: (b, 0, 0)),
        out_shape=jax.ShapeDtypeStruct((B, _NB, _QB), jnp.float32),
    )(class_logits)

    obj3 = objectness_energy.reshape(B, _NB, _QB)
    kno3 = knownness_energy.reshape(B, _NB, _QB)

    qacc, pkacc, puacc, rid, cbx = pl.pallas_call(
        _select_kernel,
        in_specs=[
            pl.BlockSpec((B, _NB, _QB), lambda: (0, 0, 0)),
            pl.BlockSpec((B, _NB, _QB), lambda: (0, 0, 0)),
            pl.BlockSpec((B, _NB, _QB), lambda: (0, 0, 0)),
            pl.BlockSpec(memory_space=pl.ANY),
        ],
        out_specs=[
            pl.BlockSpec((B, _C), lambda: (0, 0)),
            pl.BlockSpec((B, _C), lambda: (0, 0)),
            pl.BlockSpec((B, _C), lambda: (0, 0)),
            pl.BlockSpec((B, _C), lambda: (0, 0)),
            pl.BlockSpec((B, _K1, 4), lambda: (0, 0, 0)),
        ],
        out_shape=[
            jax.ShapeDtypeStruct((B, _C), jnp.int32),
            jax.ShapeDtypeStruct((B, _C), jnp.float32),
            jax.ShapeDtypeStruct((B, _C), jnp.float32),
            jax.ShapeDtypeStruct((B, _C), jnp.int32),
            jax.ShapeDtypeStruct((B, _K1, 4), jnp.float32),
        ],
        scratch_shapes=[pltpu.SemaphoreType.DMA],
    )(ml, obj3, kno3, pred_boxes)

    lgflat = class_logits.reshape(B * _Q, _C)
    gathered = pl.kernel(
        _sc_gather_kernel,
        mesh=plsc.VectorSubcoreMesh(core_axis_name="c", subcore_axis_name="s"),
        out_type=jax.ShapeDtypeStruct((_GN, _C), jnp.float32),
        scratch_types=[
            pltpu.VMEM((_GW,), jnp.int32),
            pltpu.VMEM((_GW, _C), jnp.float32),
            pltpu.SemaphoreType.DMA,
        ],
    )(rid.reshape(_GN), lgflat)

    sco, flo, bxo = pl.pallas_call(
        _finalize_kernel,
        in_specs=[
            pl.BlockSpec((B, _C, _C), lambda: (0, 0, 0)),
            pl.BlockSpec((B, _C), lambda: (0, 0)),
            pl.BlockSpec((B, _C), lambda: (0, 0)),
            pl.BlockSpec((B, _C), lambda: (0, 0)),
            pl.BlockSpec((B, _K1, 4), lambda: (0, 0, 0)),
            pl.BlockSpec((B, 2), lambda: (0, 0)),
        ],
        out_specs=[
            pl.BlockSpec((B, _C), lambda: (0, 0)),
            pl.BlockSpec((B, _C), lambda: (0, 0)),
            pl.BlockSpec((B, _C, 4), lambda: (0, 0, 0)),
        ],
        out_shape=[
            jax.ShapeDtypeStruct((B, _C), jnp.float32),
            jax.ShapeDtypeStruct((B, _C), jnp.int32),
            jax.ShapeDtypeStruct((B, _C, 4), jnp.float32),
        ],
    )(gathered.reshape(B, _C, _C), qacc, pkacc, puacc, cbx, target_sizes)

    scores = sco[:, :_K]
    flat = flo[:, :_K]
    labels = flat % _C
    boxes = bxo[:, :_K, :]
    return scores, labels, boxes


# speedup vs baseline: 2.0552x; 1.0281x over previous
import functools

import jax
import jax.numpy as jnp
from jax import lax
from jax.experimental import pallas as pl
from jax.experimental.pallas import tpu as pltpu
from jax.experimental.pallas import tpu_sc as plsc

_INVALID0, _INVALID1 = 100, 102
_C = 128
_QB = 2000
_NB = 10
_B = 8
_Q = _QB * _NB
_K1 = 104
_K = 100
_NEG = -1.0
_BIG = 2**30
_NW = 32
_GN = _B * _C
_GW = _GN // _NW


def _rowmax_kernel(lg_ref, out_ref):
    j = pl.program_id(1)
    lg = lg_ref[0]
    c = jax.lax.broadcasted_iota(jnp.int32, (_QB, _C), 1)
    valid = ((c < _INVALID0) | (c > _INVALID1)) & (c < _C - 1)
    ml = jnp.max(jnp.where(valid, lg, -1e30), axis=1)
    out_ref[0, pl.ds(j, 1), :] = ml[None, :]


def _select_kernel(ml_ref, obj_ref, kno_ref, bx_hbm,
                   q_ref, pk_ref, pu_ref, rid_ref, cbx_ref, sem2):
    ml2 = ml_ref[...]
    op = jnp.clip(jnp.exp(-obj_ref[...]), 1e-6, 1.0)
    kp = jnp.clip(jnp.exp(-kno_ref[...]), 1e-6, 1.0)
    pk2 = op * kp
    pu2 = op * jnp.clip(1.0 - kp, 0.0, 1.0) * 15.0
    m2 = jnp.maximum(pk2 * jax.nn.sigmoid(ml2), pu2)

    flat3 = (jax.lax.broadcasted_iota(jnp.int32, (_B, _NB, _QB), 1) * _QB
             + jax.lax.broadcasted_iota(jnp.int32, (_B, _NB, _QB), 2))
    lane8 = jax.lax.broadcasted_iota(jnp.int32, (_B, _C), 1)

    def body(i, carry):
        m2, qacc, pkacc, puacc = carry
        mx = jnp.max(m2, axis=(1, 2), keepdims=True)
        hit = m2 == mx
        qstar = jnp.min(jnp.where(hit, flat3, _BIG),
                        axis=(1, 2), keepdims=True)
        pick = hit & (flat3 == qstar)
        pkv = jnp.sum(jnp.where(pick, pk2, 0.0), axis=(1, 2), keepdims=True)
        puv = jnp.sum(jnp.where(pick, pu2, 0.0), axis=(1, 2), keepdims=True)
        onec = lane8 == i
        qacc = qacc + jnp.where(onec, qstar[:, :, 0], 0)
        pkacc = pkacc + jnp.where(onec, pkv[:, :, 0], 0.0)
        puacc = puacc + jnp.where(onec, puv[:, :, 0], 0.0)
        for b in range(_B):
            q = qstar[b, 0, 0]
            pltpu.make_async_copy(bx_hbm.at[b, q], cbx_ref.at[b, i],
                                  sem2).start()
        m2 = jnp.where(pick, _NEG, m2)
        return m2, qacc, pkacc, puacc

    qacc0 = jnp.zeros((_B, _C), jnp.int32)
    facc0 = jnp.zeros((_B, _C), jnp.float32)
    _, qacc, pkacc, puacc = jax.lax.fori_loop(
        0, _K1, body, (m2, qacc0, facc0, facc0))

    pltpu.make_async_copy(bx_hbm.at[:, 0:_K1, :], cbx_ref, sem2).wait()

    sub8 = jax.lax.broadcasted_iota(jnp.int32, (_B, _C), 0)
    q_ref[...] = qacc
    pk_ref[...] = pkacc
    pu_ref[...] = puacc
    rid_ref[...] = qacc + sub8 * _Q


def _sc_gather_kernel(rid_hbm, lgflat_hbm, out_hbm, idx_v, rows_v, sem):
    wid = lax.axis_index("s") * 2 + lax.axis_index("c")
    base = wid * _GW
    pltpu.sync_copy(rid_hbm.at[pl.ds(base, _GW)], idx_v)
    pltpu.async_copy(lgflat_hbm.at[idx_v], rows_v, sem).wait()
    pltpu.sync_copy(rows_v, out_hbm.at[pl.ds(base, _GW)])


def _finalize_kernel(lgc_ref, q_ref, pk_ref, pu_ref, cbx_ref, ts_ref,
                     sc_ref, fl_ref, bo_ref):
    lgc = lgc_ref[:, 0:_K1, :]
    qacc = q_ref[...]
    c3 = jax.lax.broadcasted_iota(jnp.int32, (_B, _K1, _C), 2)
    valid = ((c3 < _INVALID0) | (c3 > _INVALID1)) & (c3 < _C - 1)
    pks = pk_ref[...][:, :_K1, None]
    pus = pu_ref[...][:, :_K1, None]
    qss = qacc[:, :_K1, None]
    s = jnp.where(valid, pks * jax.nn.sigmoid(lgc), 0.0)
    s = jnp.where(c3 == _C - 1, pus, s)
    g = qss * _C + c3
    sub = jax.lax.broadcasted_iota(jnp.int32, (_B, _K1, _C), 1)
    sub_b = jax.lax.broadcasted_iota(jnp.int32, (_B, _K1, 1), 1)
    out_b = jax.lax.broadcasted_iota(jnp.int32, (_B, _C, 1), 1)
    lane8 = jax.lax.broadcasted_iota(jnp.int32, (_B, _C), 1)
    boxc = cbx_ref[...]

    def body3(t, carry):
        s, sco, flo, bxo = carry
        mx = jnp.max(s, axis=(1, 2), keepdims=True)
        hit = s == mx
        gm = jnp.min(jnp.where(hit, g, _BIG), axis=(1, 2), keepdims=True)
        pick = hit & (g == gm)
        r = jnp.min(jnp.where(pick, sub, _BIG), axis=(1, 2), keepdims=True)
        onec = lane8 == t
        sco = sco + jnp.where(onec, mx[:, :, 0], 0.0)
        flo = flo + jnp.where(onec, gm[:, :, 0], 0)
        brow = jnp.sum(jnp.where(sub_b == r, boxc, 0.0), axis=1,
                       keepdims=True)
        bxo = bxo + jnp.where(out_b == t, brow, 0.0)
        s = jnp.where(pick, _NEG, s)
        return s, sco, flo, bxo

    sco0 = jnp.zeros((_B, _C), jnp.float32)
    flo0 = jnp.zeros((_B, _C), jnp.int32)
    bxo0 = jnp.zeros((_B, _C, 4), jnp.float32)
    _, sco, flo, bxo = jax.lax.fori_loop(0, _K, body3, (s, sco0, flo0, bxo0))

    ts = ts_ref[...].astype(jnp.float32)
    hpx = ts[:, 0:1, None]
    wpx = ts[:, 1:2, None]
    cx = bxo[:, :, 0:1]
    cy = bxo[:, :, 1:2]
    w = bxo[:, :, 2:3]
    h = bxo[:, :, 3:4]
    xyxy = jnp.concatenate(
        [(cx - 0.5 * w) * wpx, (cy - 0.5 * h) * hpx,
         (cx + 0.5 * w) * wpx, (cy + 0.5 * h) * hpx], axis=2)

    sc_ref[...] = sco
    fl_ref[...] = flo
    bo_ref[...] = xyxy


@jax.jit
def kernel(class_logits, objectness_energy, knownness_energy, pred_boxes,
           target_sizes):
    B, Q, C = class_logits.shape
    assert C == _C and Q == _Q and B == _B

    ml = pl.pallas_call(
        _rowmax_kernel,
        grid=(B, _NB),
        in_specs=[pl.BlockSpec((1, _QB, _C), lambda b, j: (b, j, 0))],
        out_specs=pl.BlockSpec((1, _NB, _QB), lambda b, j: (b, 0, 0)),
        out_shape=jax.ShapeDtypeStruct((B, _NB, _QB), jnp.float32),
    )(class_logits)

    obj3 = objectness_energy.reshape(B, _NB, _QB)
    kno3 = knownness_energy.reshape(B, _NB, _QB)

    qacc, pkacc, puacc, rid, cbx = pl.pallas_call(
        _select_kernel,
        in_specs=[
            pl.BlockSpec((B, _NB, _QB), lambda: (0, 0, 0)),
            pl.BlockSpec((B, _NB, _QB), lambda: (0, 0, 0)),
            pl.BlockSpec((B, _NB, _QB), lambda: (0, 0, 0)),
            pl.BlockSpec(memory_space=pl.ANY),
        ],
        out_specs=[
            pl.BlockSpec((B, _C), lambda: (0, 0)),
            pl.BlockSpec((B, _C), lambda: (0, 0)),
            pl.BlockSpec((B, _C), lambda: (0, 0)),
            pl.BlockSpec((B, _C), lambda: (0, 0)),
            pl.BlockSpec((B, _K1, 4), lambda: (0, 0, 0)),
        ],
        out_shape=[
            jax.ShapeDtypeStruct((B, _C), jnp.int32),
            jax.ShapeDtypeStruct((B, _C), jnp.float32),
            jax.ShapeDtypeStruct((B, _C), jnp.float32),
            jax.ShapeDtypeStruct((B, _C), jnp.int32),
            jax.ShapeDtypeStruct((B, _K1, 4), jnp.float32),
        ],
        scratch_shapes=[pltpu.SemaphoreType.DMA],
    )(ml, obj3, kno3, pred_boxes)

    lgflat = class_logits.reshape(B * _Q, _C)
    gathered = pl.kernel(
        _sc_gather_kernel,
        mesh=plsc.VectorSubcoreMesh(core_axis_name="c", subcore_axis_name="s"),
        out_type=jax.ShapeDtypeStruct((_GN, _C), jnp.float32),
        scratch_types=[
            pltpu.VMEM((_GW,), jnp.int32),
            pltpu.VMEM((_GW, _C), jnp.float32),
            pltpu.SemaphoreType.DMA,
        ],
    )(rid.reshape(_GN), lgflat)

    sco, flo, bxo = pl.pallas_call(
        _finalize_kernel,
        in_specs=[
            pl.BlockSpec((B, _C, _C), lambda: (0, 0, 0)),
            pl.BlockSpec((B, _C), lambda: (0, 0)),
            pl.BlockSpec((B, _C), lambda: (0, 0)),
            pl.BlockSpec((B, _C), lambda: (0, 0)),
            pl.BlockSpec((B, _K1, 4), lambda: (0, 0, 0)),
            pl.BlockSpec((B, 2), lambda: (0, 0)),
        ],
        out_specs=[
            pl.BlockSpec((B, _C), lambda: (0, 0)),
            pl.BlockSpec((B, _C), lambda: (0, 0)),
            pl.BlockSpec((B, _C, 4), lambda: (0, 0, 0)),
        ],
        out_shape=[
            jax.ShapeDtypeStruct((B, _C), jnp.float32),
            jax.ShapeDtypeStruct((B, _C), jnp.int32),
            jax.ShapeDtypeStruct((B, _C, 4), jnp.float32),
        ],
    )(gathered.reshape(B, _C, _C), qacc, pkacc, puacc, cbx, target_sizes)

    scores = sco[:, :_K]
    flat = flo[:, :_K]
    labels = flat % _C
    boxes = bxo[:, :_K, :]
    return scores, labels, boxes


# speedup vs baseline: 2.1766x; 1.0591x over previous
import functools

import jax
import jax.numpy as jnp
from jax import lax
from jax.experimental import pallas as pl
from jax.experimental.pallas import tpu as pltpu
from jax.experimental.pallas import tpu_sc as plsc

_INVALID0, _INVALID1 = 100, 102
_C = 128
_QB = 2000
_NB = 10
_B = 8
_Q = _QB * _NB
_K1 = 104
_K = 100
_NEG = -1.0
_BIG = 2**30
_NW = 32
_GN = _B * _C
_GW = _GN // _NW


def _rowmax_kernel(lg_ref, out_ref):
    j = pl.program_id(1)
    lg = lg_ref[0]
    c = jax.lax.broadcasted_iota(jnp.int32, (_QB, _C), 1)
    valid = ((c < _INVALID0) | (c > _INVALID1)) & (c < _C - 1)
    ml = jnp.max(jnp.where(valid, lg, -1e30), axis=1)
    out_ref[0, pl.ds(j, 1), :] = ml[None, :]


def _select_kernel(ml_ref, obj_ref, kno_ref, bx_hbm,
                   q_ref, pk_ref, pu_ref, rid_ref, cbx_ref, sem2):
    ml2 = ml_ref[...]
    op = jnp.clip(jnp.exp(-obj_ref[...]), 1e-6, 1.0)
    kp = jnp.clip(jnp.exp(-kno_ref[...]), 1e-6, 1.0)
    pk2 = op * kp
    pu2 = op * jnp.clip(1.0 - kp, 0.0, 1.0) * 15.0
    m2 = jnp.maximum(pk2 * jax.nn.sigmoid(ml2), pu2)

    flat3 = (jax.lax.broadcasted_iota(jnp.int32, (_B, _NB, _QB), 1) * _QB
             + jax.lax.broadcasted_iota(jnp.int32, (_B, _NB, _QB), 2))
    lane8 = jax.lax.broadcasted_iota(jnp.int32, (_B, _C), 1)

    def body(i, carry):
        m2, qacc, pkacc, puacc = carry
        mx = jnp.max(m2, axis=(1, 2), keepdims=True)
        hit = m2 == mx
        qstar = jnp.min(jnp.where(hit, flat3, _BIG),
                        axis=(1, 2), keepdims=True)
        pick = flat3 == qstar
        pkv = jnp.sum(jnp.where(pick, pk2, 0.0), axis=(1, 2), keepdims=True)
        puv = jnp.sum(jnp.where(pick, pu2, 0.0), axis=(1, 2), keepdims=True)
        onec = lane8 == i
        qacc = qacc + jnp.where(onec, qstar[:, :, 0], 0)
        pkacc = pkacc + jnp.where(onec, pkv[:, :, 0], 0.0)
        puacc = puacc + jnp.where(onec, puv[:, :, 0], 0.0)
        for b in range(_B):
            q = qstar[b, 0, 0]
            pltpu.make_async_copy(bx_hbm.at[b, q], cbx_ref.at[b, i],
                                  sem2).start()
        m2 = jnp.where(pick, _NEG, m2)
        return m2, qacc, pkacc, puacc

    qacc0 = jnp.zeros((_B, _C), jnp.int32)
    facc0 = jnp.zeros((_B, _C), jnp.float32)
    _, qacc, pkacc, puacc = jax.lax.fori_loop(
        0, _K1, body, (m2, qacc0, facc0, facc0))

    pltpu.make_async_copy(bx_hbm.at[:, 0:_K1, :], cbx_ref, sem2).wait()

    sub8 = jax.lax.broadcasted_iota(jnp.int32, (_B, _C), 0)
    q_ref[...] = qacc
    pk_ref[...] = pkacc
    pu_ref[...] = puacc
    rid_ref[...] = qacc + sub8 * _Q


def _sc_gather_kernel(rid_hbm, lgflat_hbm, out_hbm, idx_v, rows_v, sem):
    wid = lax.axis_index("s") * 2 + lax.axis_index("c")
    base = wid * _GW
    pltpu.sync_copy(rid_hbm.at[pl.ds(base, _GW)], idx_v)
    pltpu.async_copy(lgflat_hbm.at[idx_v], rows_v, sem).wait()
    pltpu.sync_copy(rows_v, out_hbm.at[pl.ds(base, _GW)])


def _finalize_kernel(lgc_ref, q_ref, pk_ref, pu_ref, cbx_ref, ts_ref,
                     sc_ref, fl_ref, bo_ref):
    lgc = lgc_ref[:, 0:_K1, :]
    qacc = q_ref[...]
    c3 = jax.lax.broadcasted_iota(jnp.int32, (_B, _K1, _C), 2)
    valid = ((c3 < _INVALID0) | (c3 > _INVALID1)) & (c3 < _C - 1)
    pks = pk_ref[...][:, :_K1, None]
    pus = pu_ref[...][:, :_K1, None]
    qss = qacc[:, :_K1, None]
    s = jnp.where(valid, pks * jax.nn.sigmoid(lgc), 0.0)
    s = jnp.where(c3 == _C - 1, pus, s)
    g = qss * _C + c3
    sub = jax.lax.broadcasted_iota(jnp.int32, (_B, _K1, _C), 1)
    sub_b = jax.lax.broadcasted_iota(jnp.int32, (_B, _K1, 1), 1)
    out_b = jax.lax.broadcasted_iota(jnp.int32, (_B, _C, 1), 1)
    lane8 = jax.lax.broadcasted_iota(jnp.int32, (_B, _C), 1)
    boxc = cbx_ref[...]

    def body3(t, carry):
        s, sco, flo, bxo = carry
        mx = jnp.max(s, axis=(1, 2), keepdims=True)
        hit = s == mx
        gm = jnp.min(jnp.where(hit, g, _BIG), axis=(1, 2), keepdims=True)
        pick = g == gm
        r = jnp.min(jnp.where(pick, sub, _BIG), axis=(1, 2), keepdims=True)
        onec = lane8 == t
        sco = sco + jnp.where(onec, mx[:, :, 0], 0.0)
        flo = flo + jnp.where(onec, gm[:, :, 0], 0)
        brow = jnp.sum(jnp.where(sub_b == r, boxc, 0.0), axis=1,
                       keepdims=True)
        bxo = bxo + jnp.where(out_b == t, brow, 0.0)
        s = jnp.where(pick, _NEG, s)
        return s, sco, flo, bxo

    sco0 = jnp.zeros((_B, _C), jnp.float32)
    flo0 = jnp.zeros((_B, _C), jnp.int32)
    bxo0 = jnp.zeros((_B, _C, 4), jnp.float32)
    _, sco, flo, bxo = jax.lax.fori_loop(0, _K, body3, (s, sco0, flo0, bxo0))

    ts = ts_ref[...].astype(jnp.float32)
    hpx = ts[:, 0:1, None]
    wpx = ts[:, 1:2, None]
    cx = bxo[:, :, 0:1]
    cy = bxo[:, :, 1:2]
    w = bxo[:, :, 2:3]
    h = bxo[:, :, 3:4]
    xyxy = jnp.concatenate(
        [(cx - 0.5 * w) * wpx, (cy - 0.5 * h) * hpx,
         (cx + 0.5 * w) * wpx, (cy + 0.5 * h) * hpx], axis=2)

    sc_ref[...] = sco
    fl_ref[...] = flo
    bo_ref[...] = xyxy


@jax.jit
def kernel(class_logits, objectness_energy, knownness_energy, pred_boxes,
           target_sizes):
    B, Q, C = class_logits.shape
    assert C == _C and Q == _Q and B == _B

    ml = pl.pallas_call(
        _rowmax_kernel,
        grid=(B, _NB),
        in_specs=[pl.BlockSpec((1, _QB, _C), lambda b, j: (b, j, 0))],
        out_specs=pl.BlockSpec((1, _NB, _QB), lambda b, j: (b, 0, 0)),
        out_shape=jax.ShapeDtypeStruct((B, _NB, _QB), jnp.float32),
    )(class_logits)

    obj3 = objectness_energy.reshape(B, _NB, _QB)
    kno3 = knownness_energy.reshape(B, _NB, _QB)

    qacc, pkacc, puacc, rid, cbx = pl.pallas_call(
        _select_kernel,
        in_specs=[
            pl.BlockSpec((B, _NB, _QB), lambda: (0, 0, 0)),
            pl.BlockSpec((B, _NB, _QB), lambda: (0, 0, 0)),
            pl.BlockSpec((B, _NB, _QB), lambda: (0, 0, 0)),
            pl.BlockSpec(memory_space=pl.ANY),
        ],
        out_specs=[
            pl.BlockSpec((B, _C), lambda: (0, 0)),
            pl.BlockSpec((B, _C), lambda: (0, 0)),
            pl.BlockSpec((B, _C), lambda: (0, 0)),
            pl.BlockSpec((B, _C), lambda: (0, 0)),
            pl.BlockSpec((B, _K1, 4), lambda: (0, 0, 0)),
        ],
        out_shape=[
            jax.ShapeDtypeStruct((B, _C), jnp.int32),
            jax.ShapeDtypeStruct((B, _C), jnp.float32),
            jax.ShapeDtypeStruct((B, _C), jnp.float32),
            jax.ShapeDtypeStruct((B, _C), jnp.int32),
            jax.ShapeDtypeStruct((B, _K1, 4), jnp.float32),
        ],
        scratch_shapes=[pltpu.SemaphoreType.DMA],
    )(ml, obj3, kno3, pred_boxes)

    lgflat = class_logits.reshape(B * _Q, _C)
    gathered = pl.kernel(
        _sc_gather_kernel,
        mesh=plsc.VectorSubcoreMesh(core_axis_name="c", subcore_axis_name="s"),
        out_type=jax.ShapeDtypeStruct((_GN, _C), jnp.float32),
        scratch_types=[
            pltpu.VMEM((_GW,), jnp.int32),
            pltpu.VMEM((_GW, _C), jnp.float32),
            pltpu.SemaphoreType.DMA,
        ],
    )(rid.reshape(_GN), lgflat)

    sco, flo, bxo = pl.pallas_call(
        _finalize_kernel,
        in_specs=[
            pl.BlockSpec((B, _C, _C), lambda: (0, 0, 0)),
            pl.BlockSpec((B, _C), lambda: (0, 0)),
            pl.BlockSpec((B, _C), lambda: (0, 0)),
            pl.BlockSpec((B, _C), lambda: (0, 0)),
            pl.BlockSpec((B, _K1, 4), lambda: (0, 0, 0)),
            pl.BlockSpec((B, 2), lambda: (0, 0)),
        ],
        out_specs=[
            pl.BlockSpec((B, _C), lambda: (0, 0)),
            pl.BlockSpec((B, _C), lambda: (0, 0)),
            pl.BlockSpec((B, _C, 4), lambda: (0, 0, 0)),
        ],
        out_shape=[
            jax.ShapeDtypeStruct((B, _C), jnp.float32),
            jax.ShapeDtypeStruct((B, _C), jnp.int32),
            jax.ShapeDtypeStruct((B, _C, 4), jnp.float32),
        ],
    )(gathered.reshape(B, _C, _C), qacc, pkacc, puacc, cbx, target_sizes)

    scores = sco[:, :_K]
    flat = flo[:, :_K]
    labels = flat % _C
    boxes = bxo[:, :_K, :]
    return scores, labels, boxes


# speedup vs baseline: 2.5527x; 1.1728x over previous
import functools

import jax
import jax.numpy as jnp
from jax import lax
from jax.experimental import pallas as pl
from jax.experimental.pallas import tpu as pltpu
from jax.experimental.pallas import tpu_sc as plsc

_INVALID0, _INVALID1 = 100, 102
_C = 128
_QB = 2000
_NB = 10
_B = 8
_Q = _QB * _NB
_K1 = 104
_K = 100
_NEG = -1.0
_BIG = 2**30
_NW = 32
_GN = _B * _C
_GW = _GN // _NW


_SR = 16
_NR = _NB * _SR


def _rowmax_kernel(lg_ref, out_ref):
    j = pl.program_id(1)
    lg = lg_ref[0]
    c = jax.lax.broadcasted_iota(jnp.int32, (_QB, _C), 1)
    valid = ((c < _INVALID0) | (c > _INVALID1)) & (c < _C - 1)
    ml = jnp.max(jnp.where(valid, lg, -1e30), axis=1, keepdims=True)
    mlp = jnp.concatenate(
        [ml, jnp.full((_SR * _C - _QB, 1), -1e30, jnp.float32)], axis=0)
    out_ref[0, pl.ds(j * _SR, _SR), :] = mlp.reshape(_SR, _C)


def _select_kernel(ml_ref, obj_ref, kno_ref, bx_hbm,
                   q_ref, pk_ref, pu_ref, rid_ref, cbx_ref, sem2):
    ml2 = ml_ref[...]
    op = jnp.clip(jnp.exp(-obj_ref[...]), 1e-6, 1.0)
    kp = jnp.clip(jnp.exp(-kno_ref[...]), 1e-6, 1.0)
    pk2 = op * kp
    pu2 = op * jnp.clip(1.0 - kp, 0.0, 1.0) * 15.0
    m2 = jnp.maximum(pk2 * jax.nn.sigmoid(ml2), pu2)

    rowp = jax.lax.broadcasted_iota(jnp.int32, (_B, _NR, _C), 1)
    lanep = jax.lax.broadcasted_iota(jnp.int32, (_B, _NR, _C), 2)
    pad = (rowp % _SR == _SR - 1) & (lanep >= _QB - (_SR - 1) * _C)
    m2 = jnp.where(pad, _NEG, m2)
    flat3 = ((rowp // _SR) * _QB + (rowp % _SR) * _C + lanep)
    flat3 = jnp.where(pad, 2**20 + rowp * _C + lanep, flat3)
    lane8 = jax.lax.broadcasted_iota(jnp.int32, (_B, _C), 1)

    def body(i, carry):
        m2, qacc, pkacc, puacc = carry
        mx = jnp.max(m2, axis=(1, 2), keepdims=True)
        hit = m2 == mx
        qstar = jnp.min(jnp.where(hit, flat3, _BIG),
                        axis=(1, 2), keepdims=True)
        pick = flat3 == qstar
        pkv = jnp.sum(jnp.where(pick, pk2, 0.0), axis=(1, 2), keepdims=True)
        puv = jnp.sum(jnp.where(pick, pu2, 0.0), axis=(1, 2), keepdims=True)
        onec = lane8 == i
        qacc = qacc + jnp.where(onec, qstar[:, :, 0], 0)
        pkacc = pkacc + jnp.where(onec, pkv[:, :, 0], 0.0)
        puacc = puacc + jnp.where(onec, puv[:, :, 0], 0.0)
        for b in range(_B):
            q = qstar[b, 0, 0]
            pltpu.make_async_copy(bx_hbm.at[b, q], cbx_ref.at[b, i],
                                  sem2).start()
        m2 = jnp.where(pick, _NEG, m2)
        return m2, qacc, pkacc, puacc

    qacc0 = jnp.zeros((_B, _C), jnp.int32)
    facc0 = jnp.zeros((_B, _C), jnp.float32)
    _, qacc, pkacc, puacc = jax.lax.fori_loop(
        0, _K1, body, (m2, qacc0, facc0, facc0))

    pltpu.make_async_copy(bx_hbm.at[:, 0:_K1, :], cbx_ref, sem2).wait()

    sub8 = jax.lax.broadcasted_iota(jnp.int32, (_B, _C), 0)
    q_ref[...] = qacc
    pk_ref[...] = pkacc
    pu_ref[...] = puacc
    rid_ref[...] = qacc + sub8 * _Q


def _sc_gather_kernel(rid_hbm, lgflat_hbm, out_hbm, idx_v, rows_v, sem):
    wid = lax.axis_index("s") * 2 + lax.axis_index("c")
    base = wid * _GW
    pltpu.sync_copy(rid_hbm.at[pl.ds(base, _GW)], idx_v)
    pltpu.async_copy(lgflat_hbm.at[idx_v], rows_v, sem).wait()
    pltpu.sync_copy(rows_v, out_hbm.at[pl.ds(base, _GW)])


def _finalize_kernel(lgc_ref, q_ref, pk_ref, pu_ref, cbx_ref, ts_ref,
                     sc_ref, fl_ref, bo_ref):
    lgc = lgc_ref[:, 0:_K1, :]
    qacc = q_ref[...]
    c3 = jax.lax.broadcasted_iota(jnp.int32, (_B, _K1, _C), 2)
    valid = ((c3 < _INVALID0) | (c3 > _INVALID1)) & (c3 < _C - 1)
    pks = pk_ref[...][:, :_K1, None]
    pus = pu_ref[...][:, :_K1, None]
    qss = qacc[:, :_K1, None]
    s = jnp.where(valid, pks * jax.nn.sigmoid(lgc), 0.0)
    s = jnp.where(c3 == _C - 1, pus, s)
    g = qss * _C + c3
    sub = jax.lax.broadcasted_iota(jnp.int32, (_B, _K1, _C), 1)
    sub_b = jax.lax.broadcasted_iota(jnp.int32, (_B, _K1, 1), 1)
    out_b = jax.lax.broadcasted_iota(jnp.int32, (_B, _C, 1), 1)
    lane8 = jax.lax.broadcasted_iota(jnp.int32, (_B, _C), 1)
    boxc = cbx_ref[...]

    def body3(t, carry):
        s, sco, flo, bxo = carry
        mx = jnp.max(s, axis=(1, 2), keepdims=True)
        hit = s == mx
        gm = jnp.min(jnp.where(hit, g, _BIG), axis=(1, 2), keepdims=True)
        pick = g == gm
        r = jnp.min(jnp.where(pick, sub, _BIG), axis=(1, 2), keepdims=True)
        onec = lane8 == t
        sco = sco + jnp.where(onec, mx[:, :, 0], 0.0)
        flo = flo + jnp.where(onec, gm[:, :, 0], 0)
        brow = jnp.sum(jnp.where(sub_b == r, boxc, 0.0), axis=1,
                       keepdims=True)
        bxo = bxo + jnp.where(out_b == t, brow, 0.0)
        s = jnp.where(pick, _NEG, s)
        return s, sco, flo, bxo

    sco0 = jnp.zeros((_B, _C), jnp.float32)
    flo0 = jnp.zeros((_B, _C), jnp.int32)
    bxo0 = jnp.zeros((_B, _C, 4), jnp.float32)
    _, sco, flo, bxo = jax.lax.fori_loop(0, _K, body3, (s, sco0, flo0, bxo0))

    ts = ts_ref[...].astype(jnp.float32)
    hpx = ts[:, 0:1, None]
    wpx = ts[:, 1:2, None]
    cx = bxo[:, :, 0:1]
    cy = bxo[:, :, 1:2]
    w = bxo[:, :, 2:3]
    h = bxo[:, :, 3:4]
    xyxy = jnp.concatenate(
        [(cx - 0.5 * w) * wpx, (cy - 0.5 * h) * hpx,
         (cx + 0.5 * w) * wpx, (cy + 0.5 * h) * hpx], axis=2)

    sc_ref[...] = sco
    fl_ref[...] = flo
    bo_ref[...] = xyxy


def _pad_energy(e):
    e3 = e.reshape(_B, _NB, _QB)
    e3 = jnp.pad(e3, ((0, 0), (0, 0), (0, _SR * _C - _QB)))
    return e3.reshape(_B, _NR, _C)


@jax.jit
def kernel(class_logits, objectness_energy, knownness_energy, pred_boxes,
           target_sizes):
    B, Q, C = class_logits.shape
    assert C == _C and Q == _Q and B == _B

    ml = pl.pallas_call(
        _rowmax_kernel,
        grid=(B, _NB),
        in_specs=[pl.BlockSpec((1, _QB, _C), lambda b, j: (b, j, 0))],
        out_specs=pl.BlockSpec((1, _NR, _C), lambda b, j: (b, 0, 0)),
        out_shape=jax.ShapeDtypeStruct((B, _NR, _C), jnp.float32),
    )(class_logits)

    obj3 = _pad_energy(objectness_energy)
    kno3 = _pad_energy(knownness_energy)

    qacc, pkacc, puacc, rid, cbx = pl.pallas_call(
        _select_kernel,
        in_specs=[
            pl.BlockSpec((B, _NR, _C), lambda: (0, 0, 0)),
            pl.BlockSpec((B, _NR, _C), lambda: (0, 0, 0)),
            pl.BlockSpec((B, _NR, _C), lambda: (0, 0, 0)),
            pl.BlockSpec(memory_space=pl.ANY),
        ],
        out_specs=[
            pl.BlockSpec((B, _C), lambda: (0, 0)),
            pl.BlockSpec((B, _C), lambda: (0, 0)),
            pl.BlockSpec((B, _C), lambda: (0, 0)),
            pl.BlockSpec((B, _C), lambda: (0, 0)),
            pl.BlockSpec((B, _K1, 4), lambda: (0, 0, 0)),
        ],
        out_shape=[
            jax.ShapeDtypeStruct((B, _C), jnp.int32),
            jax.ShapeDtypeStruct((B, _C), jnp.float32),
            jax.ShapeDtypeStruct((B, _C), jnp.float32),
            jax.ShapeDtypeStruct((B, _C), jnp.int32),
            jax.ShapeDtypeStruct((B, _K1, 4), jnp.float32),
        ],
        scratch_shapes=[pltpu.SemaphoreType.DMA],
    )(ml, obj3, kno3, pred_boxes)

    lgflat = class_logits.reshape(B * _Q, _C)
    gathered = pl.kernel(
        _sc_gather_kernel,
        mesh=plsc.VectorSubcoreMesh(core_axis_name="c", subcore_axis_name="s"),
        out_type=jax.ShapeDtypeStruct((_GN, _C), jnp.float32),
        scratch_types=[
            pltpu.VMEM((_GW,), jnp.int32),
            pltpu.VMEM((_GW, _C), jnp.float32),
            pltpu.SemaphoreType.DMA,
        ],
    )(rid.reshape(_GN), lgflat)

    sco, flo, bxo = pl.pallas_call(
        _finalize_kernel,
        in_specs=[
            pl.BlockSpec((B, _C, _C), lambda: (0, 0, 0)),
            pl.BlockSpec((B, _C), lambda: (0, 0)),
            pl.BlockSpec((B, _C), lambda: (0, 0)),
            pl.BlockSpec((B, _C), lambda: (0, 0)),
            pl.BlockSpec((B, _K1, 4), lambda: (0, 0, 0)),
            pl.BlockSpec((B, 2), lambda: (0, 0)),
        ],
        out_specs=[
            pl.BlockSpec((B, _C), lambda: (0, 0)),
            pl.BlockSpec((B, _C), lambda: (0, 0)),
            pl.BlockSpec((B, _C, 4), lambda: (0, 0, 0)),
        ],
        out_shape=[
            jax.ShapeDtypeStruct((B, _C), jnp.float32),
            jax.ShapeDtypeStruct((B, _C), jnp.int32),
            jax.ShapeDtypeStruct((B, _C, 4), jnp.float32),
        ],
    )(gathered.reshape(B, _C, _C), qacc, pkacc, puacc, cbx, target_sizes)

    scores = sco[:, :_K]
    flat = flo[:, :_K]
    labels = flat % _C
    boxes = bxo[:, :_K, :]
    return scores, labels, boxes
